# baseline (speedup 1.0000x reference)
import functools

import jax
import jax.numpy as jnp
from jax import lax
from jax.experimental import pallas as pl
from jax.experimental.pallas import tpu as pltpu
from jax.experimental.pallas import tpu_sc as plsc

D = 768
B, S = 4, 8192
NTOK = B * S
NC, NS, L = 2, 16, 16
NW = NC * NS
TOK_PER = NTOK // NW
CHUNK = 16
NCHUNK = TOK_PER // CHUNK
NGRP = D // (2 * L)
DW = D // 2
ROWS_A = 24 * 7
ROWS_B = 31 * 12
MASK_HI = -65536


def _tc_combine():
    def body(he, ho, we, wo, de, dd, me, mo, a_ref, b_ref):
        def pack(lo, hi):
            lo_i = lax.bitcast_convert_type(
                lo.astype(jnp.bfloat16).astype(jnp.float32), jnp.int32
            )
            hi_i = lax.bitcast_convert_type(
                hi.astype(jnp.bfloat16).astype(jnp.float32), jnp.int32
            )
            return lax.shift_right_logical(lo_i, 16) | hi_i

        a_ref[...] = pack(
            he[...][:, None, :] + we[...][None, :, :],
            ho[...][:, None, :] + wo[...][None, :, :],
        ).reshape(ROWS_A, DW)
        b_ref[...] = pack(
            de[...][:, None, :] + me[...][None, :, :],
            dd[...][:, None, :] + mo[...][None, :, :],
        ).reshape(ROWS_B, DW)

    return pl.pallas_call(
        body,
        out_shape=[
            jax.ShapeDtypeStruct((ROWS_A, DW), jnp.int32),
            jax.ShapeDtypeStruct((ROWS_B, DW), jnp.int32),
        ],
    )


_combine_tables = _tc_combine()


def _make_sc_kernel():
    mesh = plsc.VectorSubcoreMesh(core_axis_name="c", subcore_axis_name="s")

    @functools.partial(
        pl.kernel,
        mesh=mesh,
        out_type=jax.ShapeDtypeStruct((NTOK, D), jnp.float32),
        scratch_types=[
            pltpu.VMEM((4, TOK_PER), jnp.int32),
            pltpu.VMEM((2, TOK_PER), jnp.int32),
            pltpu.VMEM((2, CHUNK, DW), jnp.int32),
            pltpu.VMEM((2, CHUNK, DW), jnp.int32),
            pltpu.VMEM((2, CHUNK, D), jnp.float32),
            pltpu.SemaphoreType.DMA,
            pltpu.SemaphoreType.DMA,
            pltpu.SemaphoreType.DMA,
            pltpu.SemaphoreType.DMA,
        ],
    )
    def body(
        xt_hbm, a_hbm, b_hbm, out_hbm,
        idx_v, rows_v, buf_a, buf_b, stage,
        sg0, sg1, so0, so1,
    ):
        wid = lax.axis_index("s") * NC + lax.axis_index("c")
        base = wid * TOK_PER
        for c in range(4):
            pltpu.sync_copy(xt_hbm.at[c, pl.ds(base, TOK_PER)], idx_v.at[c])

        sgs = (sg0, sg1)
        sos = (so0, so1)

        def bc_f32(v):
            return lax.bitcast_convert_type(v, jnp.float32)

        @plsc.parallel_loop(0, NCHUNK, unroll=4)
        def prep(i):
            t = pl.ds(i * L, L)
            rows_v[0, t] = idx_v[3, t] * 7 + idx_v[2, t]
            rows_v[1, t] = idx_v[1, t] * 12 + idx_v[0, t] - 13

        def fire(ci, pb):
            t = pl.ds(ci * CHUNK, L)
            pltpu.make_async_copy(
                a_hbm.at[rows_v[0, t]], buf_a.at[pb], sgs[pb]
            ).start()
            pltpu.make_async_copy(
                b_hbm.at[rows_v[1, t]], buf_b.at[pb], sgs[pb]
            ).start()

        def drain_gather(pb):
            zeros = jnp.zeros((L,), jnp.int32)
            pltpu.make_async_copy(
                a_hbm.at[zeros], buf_a.at[pb], sgs[pb]
            ).wait()
            pltpu.make_async_copy(
                b_hbm.at[zeros], buf_b.at[pb], sgs[pb]
            ).wait()

        fire(0, 0)

        def pair_body(pi, carry):
            for pb in range(2):
                ci = pi * 2 + pb

                @pl.when(ci + 1 < NCHUNK)
                def _(ci=ci, pb=pb):
                    fire(ci + 1, 1 - pb)

                drain_gather(pb)

                for tl in range(CHUNK):

                    @plsc.parallel_loop(0, NGRP, unroll=8)
                    def grp(g, tl=tl, pb=pb):
                        col = pl.ds(g * L, L)
                        a = buf_a[pb, tl, col]
                        bb = buf_b[pb, tl, col]
                        stage[pb, tl, pl.ds(g * 2 * L, L)] = bc_f32(
                            a << 16
                        ) + bc_f32(bb << 16)
                        stage[pb, tl, pl.ds(g * 2 * L + L, L)] = bc_f32(
                            a
                        ) + bc_f32(bb)

                @pl.when(pi > 0)
                def _(pb=pb):
                    pltpu.make_async_copy(
                        stage.at[pb], out_hbm.at[pl.ds(base, CHUNK)], sos[pb]
                    ).wait()

                pltpu.make_async_copy(
                    stage.at[pb],
                    out_hbm.at[pl.ds(base + ci * CHUNK, CHUNK)],
                    sos[pb],
                ).start()
            return carry

        lax.fori_loop(0, NCHUNK // 2, pair_body, 0)
        for pb in range(2):
            pltpu.make_async_copy(
                stage.at[pb], out_hbm.at[pl.ds(base, CHUNK)], sos[pb]
            ).wait()

    return body


_sc_lookup = _make_sc_kernel()


def _halves(t):
    r = t.shape[0]
    t4 = t.reshape(r, NGRP, 2, L)
    return t4[:, :, 0, :].reshape(r, DW), t4[:, :, 1, :].reshape(r, DW)


def kernel(x, hour_w, weekday_w, day_w, month_w):
    xt = x.astype(jnp.int32).reshape(NTOK, 4).T
    he, ho = _halves(hour_w)
    we, wo = _halves(weekday_w)
    de, dd = _halves(day_w)
    me, mo = _halves(month_w)
    a_tab, b_tab = _combine_tables(he, ho, we, wo, de, dd, me, mo)
    out = _sc_lookup(xt, a_tab, b_tab)
    return out.reshape(B, S, D)


# speedup vs baseline: 10.3700x; 10.3700x over previous
import functools

import jax
import jax.numpy as jnp
from jax import lax
from jax.experimental import pallas as pl
from jax.experimental.pallas import tpu as pltpu
from jax.experimental.pallas import tpu_sc as plsc

D = 768
B, S = 4, 8192
NTOK = B * S
NC, NS, L = 2, 16, 16
NW = NC * NS
TOK_PER = NTOK // NW
CHUNK = 16
NCHUNK = TOK_PER // CHUNK
NGRP = D // (2 * L)
DW = D // 2
NROWS_HD = 56
NROWS_SRC = 24
NROWS = NROWS_HD + 7 * 12
OFF_D, OFF_WM = 24, NROWS_HD
MASK_HI = -65536


def _make_sc_kernel():
    mesh = plsc.VectorSubcoreMesh(core_axis_name="c", subcore_axis_name="s")

    @functools.partial(
        pl.kernel,
        mesh=mesh,
        out_type=jax.ShapeDtypeStruct((NTOK, D), jnp.float32),
        scratch_types=[
            pltpu.VMEM((4, TOK_PER), jnp.int32),
            pltpu.VMEM((NROWS, DW), jnp.int32),
            pltpu.VMEM((NROWS_SRC, DW), jnp.int32),
            pltpu.VMEM((2, CHUNK, D), jnp.float32),
            pltpu.SemaphoreType.DMA,
            pltpu.SemaphoreType.DMA,
        ],
    )
    def body(
        xt_hbm, hd_hbm, src_hbm, out_hbm, idx_v, tab_v, src_v, stage, sem0, sem1
    ):
        wid = lax.axis_index("s") * NC + lax.axis_index("c")
        base = wid * TOK_PER
        pltpu.sync_copy(hd_hbm, tab_v.at[pl.ds(0, NROWS_HD)])
        pltpu.sync_copy(src_hbm, src_v)
        for c in range(4):
            pltpu.sync_copy(xt_hbm.at[c, pl.ds(base, TOK_PER)], idx_v.at[c])

        sems = (sem0, sem1)

        def bc_f32(v):
            return lax.bitcast_convert_type(v, jnp.float32)

        def bc_i32(v):
            return lax.bitcast_convert_type(v, jnp.int32)

        def build_w(w_i, carry):
            def build_m(m_i, carry2):
                row = OFF_WM + w_i * 12 + m_i

                @plsc.parallel_loop(0, NGRP, unroll=4)
                def build_grp(g):
                    col = pl.ds(g * L, L)
                    a = src_v[w_i, col]
                    bm = src_v[7 + m_i, col]
                    lo = bc_f32(a << 16) + bc_f32(bm << 16)
                    hi = bc_f32(a & MASK_HI) + bc_f32(bm & MASK_HI)
                    tab_v[row, col] = (
                        lax.shift_right_logical(bc_i32(lo), 16)
                        | (bc_i32(hi) & MASK_HI)
                    )

                return carry2

            lax.fori_loop(0, 12, build_m, 0)
            return carry

        lax.fori_loop(0, 7, build_w, 0)

        def compute_chunk(ci, b):
            tok = pl.ds(ci * CHUNK, L)
            rwm = idx_v[2, tok] * 12 + idx_v[0, tok] + (OFF_WM - 1)
            rd = idx_v[1, tok] + (OFF_D - 1)
            rh = idx_v[3, tok]

            for tl0 in range(CHUNK):
                tl = tl0

                @plsc.parallel_loop(0, NGRP, unroll=8)
                def dim_body(g, tl=tl, tl0=tl0):
                    col = pl.ds(g * L, L)

                    def row(r):
                        w = tab_v[r, col]
                        return bc_f32(w << 16), bc_f32(w)

                    h_lo, h_hi = row(rh[tl])
                    d_lo, d_hi = row(rd[tl])
                    wm_lo, wm_hi = row(rwm[tl])
                    stage[b, tl0, pl.ds(g * 2 * L, L)] = (h_lo + d_lo) + wm_lo
                    stage[b, tl0, pl.ds(g * 2 * L + L, L)] = (
                        h_hi + d_hi
                    ) + wm_hi

        def pair_body(pi, carry):
            for b in range(2):
                ci = pi * 2 + b

                @pl.when(pi > 0)
                def _(b=b):
                    pltpu.make_async_copy(
                        stage.at[b], out_hbm.at[pl.ds(base, CHUNK)], sems[b]
                    ).wait()

                compute_chunk(ci, b)
                pltpu.make_async_copy(
                    stage.at[b],
                    out_hbm.at[pl.ds(base + ci * CHUNK, CHUNK)],
                    sems[b],
                ).start()
            return carry

        lax.fori_loop(0, NCHUNK // 2, pair_body, 0)
        for b in range(2):
            pltpu.make_async_copy(
                stage.at[b], out_hbm.at[pl.ds(base, CHUNK)], sems[b]
            ).wait()

    return body


_sc_lookup = _make_sc_kernel()


def _interleave_cols(t):
    r = t.shape[0]
    return t.reshape(r, NGRP, 2, L).transpose(0, 1, 3, 2).reshape(r, D)


def _pack(t, nrows):
    pad = nrows - t.shape[0]
    t = jnp.concatenate([t, jnp.zeros((pad, D), t.dtype)], axis=0)
    t = _interleave_cols(t).astype(jnp.bfloat16)
    return jax.lax.bitcast_convert_type(t.reshape(nrows, DW, 2), jnp.int32)


def kernel(x, hour_w, weekday_w, day_w, month_w):
    xt = x.astype(jnp.int32).reshape(NTOK, 4).T
    hd = _pack(jnp.concatenate([hour_w, day_w], axis=0), NROWS_HD)
    src = _pack(jnp.concatenate([weekday_w, month_w], axis=0), NROWS_SRC)
    out = _sc_lookup(xt, hd, src)
    return out.reshape(B, S, D)


# speedup vs baseline: 15.7175x; 1.5157x over previous
import functools

import jax
import jax.numpy as jnp
from jax import lax
from jax.experimental import pallas as pl
from jax.experimental.pallas import tpu as pltpu
from jax.experimental.pallas import tpu_sc as plsc

D = 768
B, S = 4, 8192
NTOK = B * S
NC, NS, L = 2, 16, 16
NW = NC * NS
TOK_PER = NTOK // NW
CHUNK = 8
NCHUNK = TOK_PER // CHUNK
NGRP = D // (2 * L)
DW = D // 2
NROWS_HD = 56
NROWS_SRC = 24
NROWS = NROWS_HD + 7 * 12
OFF_D, OFF_WM = 24, NROWS_HD
MASK_HI = -65536


def _make_sc_kernel():
    mesh = plsc.VectorSubcoreMesh(core_axis_name="c", subcore_axis_name="s")

    @functools.partial(
        pl.kernel,
        mesh=mesh,
        out_type=jax.ShapeDtypeStruct((NTOK, D), jnp.float32),
        scratch_types=[
            pltpu.VMEM((4, TOK_PER), jnp.int32),
            pltpu.VMEM((NROWS, DW), jnp.int32),
            pltpu.VMEM((NROWS_SRC, DW), jnp.int32),
            pltpu.VMEM((2, CHUNK, D), jnp.float32),
            pltpu.SemaphoreType.DMA,
            pltpu.SemaphoreType.DMA,
        ],
    )
    def body(
        xt_hbm, hd_hbm, src_hbm, out_hbm, idx_v, tab_v, src_v, stage, sem0, sem1
    ):
        wid = lax.axis_index("s") * NC + lax.axis_index("c")
        base = wid * TOK_PER
        pltpu.sync_copy(hd_hbm, tab_v.at[pl.ds(0, NROWS_HD)])
        pltpu.sync_copy(src_hbm, src_v)
        for c in range(4):
            pltpu.sync_copy(xt_hbm.at[c, pl.ds(base, TOK_PER)], idx_v.at[c])

        sems = (sem0, sem1)

        def bc_f32(v):
            return lax.bitcast_convert_type(v, jnp.float32)

        def bc_i32(v):
            return lax.bitcast_convert_type(v, jnp.int32)

        def build_w(w_i, carry):
            def build_m(m_i, carry2):
                row = OFF_WM + w_i * 12 + m_i

                @plsc.parallel_loop(0, NGRP, unroll=4)
                def build_grp(g):
                    col = pl.ds(g * L, L)
                    a = src_v[w_i, col]
                    bm = src_v[7 + m_i, col]
                    lo = bc_f32(a << 16) + bc_f32(bm << 16)
                    hi = bc_f32(a & MASK_HI) + bc_f32(bm & MASK_HI)
                    tab_v[row, col] = (
                        lax.shift_right_logical(bc_i32(lo), 16)
                        | (bc_i32(hi) & MASK_HI)
                    )

                return carry2

            lax.fori_loop(0, 12, build_m, 0)
            return carry

        lax.fori_loop(0, 7, build_w, 0)

        def compute_chunk(pi, b):
            tok = pl.ds(pi * 2 * CHUNK, L)
            rwm = idx_v[2, tok] * 12 + idx_v[0, tok] + (OFF_WM - 1)
            rd = idx_v[1, tok] + (OFF_D - 1)
            rh = idx_v[3, tok]

            for tl0 in range(CHUNK):
                tl = b * CHUNK + tl0

                @plsc.parallel_loop(0, NGRP, unroll=4)
                def dim_body(g, tl=tl, tl0=tl0):
                    col = pl.ds(g * L, L)

                    def row(r):
                        w = tab_v[r, col]
                        return bc_f32(w << 16), bc_f32(w)

                    h_lo, h_hi = row(rh[tl])
                    d_lo, d_hi = row(rd[tl])
                    wm_lo, wm_hi = row(rwm[tl])
                    stage[b, tl0, pl.ds(g * 2 * L, L)] = (h_lo + d_lo) + wm_lo
                    stage[b, tl0, pl.ds(g * 2 * L + L, L)] = (
                        h_hi + d_hi
                    ) + wm_hi

        def pair_body(pi, carry):
            for b in range(2):
                ci = pi * 2 + b

                @pl.when(pi > 0)
                def _(b=b):
                    pltpu.make_async_copy(
                        stage.at[b], out_hbm.at[pl.ds(base, CHUNK)], sems[b]
                    ).wait()

                compute_chunk(pi, b)
                pltpu.make_async_copy(
                    stage.at[b],
                    out_hbm.at[pl.ds(base + ci * CHUNK, CHUNK)],
                    sems[b],
                ).start()
            return carry

        lax.fori_loop(0, NCHUNK // 2, pair_body, 0)
        for b in range(2):
            pltpu.make_async_copy(
                stage.at[b], out_hbm.at[pl.ds(base, CHUNK)], sems[b]
            ).wait()

    return body


_sc_lookup = _make_sc_kernel()


def _interleave_cols(t):
    r = t.shape[0]
    return t.reshape(r, NGRP, 2, L).transpose(0, 1, 3, 2).reshape(r, D)


def _pack(t, nrows):
    pad = nrows - t.shape[0]
    t = jnp.concatenate([t, jnp.zeros((pad, D), t.dtype)], axis=0)
    t = _interleave_cols(t).astype(jnp.bfloat16)
    return jax.lax.bitcast_convert_type(t.reshape(nrows, DW, 2), jnp.int32)


def kernel(x, hour_w, weekday_w, day_w, month_w):
    xt = x.astype(jnp.int32).reshape(NTOK, 4).T
    hd = _pack(jnp.concatenate([hour_w, day_w], axis=0), NROWS_HD)
    src = _pack(jnp.concatenate([weekday_w, month_w], axis=0), NROWS_SRC)
    out = _sc_lookup(xt, hd, src)
    return out.reshape(B, S, D)
